# baseline (speedup 1.0000x reference)
import functools

import jax
import jax.numpy as jnp
from jax import lax
from jax.experimental import pallas as pl
from jax.experimental.pallas import tpu as pltpu
from jax.experimental.pallas import tpu_sc as plsc

D_MODEL = 768
MAX_POS = 1024
BATCH = 64
SEQ = 1024

NC = 2
NS = 16
NW = NC * NS

W = SEQ // NW
LANES = 16
DSUB = D_MODEL // LANES
NBUF = 4

_mesh = plsc.VectorSubcoreMesh(core_axis_name="c", subcore_axis_name="s")


@functools.partial(
    pl.kernel,
    mesh=_mesh,
    out_type=jax.ShapeDtypeStruct((BATCH * SEQ, D_MODEL), jnp.float32),
    scratch_types=[
        pltpu.VMEM((BATCH, W), jnp.int32),
        pltpu.VMEM((W, D_MODEL), jnp.float32),
        pltpu.VMEM((W, D_MODEL), jnp.float32),
        pltpu.VMEM((W, D_MODEL), jnp.float32),
        pltpu.VMEM((W, D_MODEL), jnp.float32),
        pltpu.VMEM((W, D_MODEL), jnp.float32),
        pltpu.SemaphoreType.DMA((NBUF,)),
        pltpu.SemaphoreType.DMA((NBUF,)),
    ],
)
def _embed(x_hbm, wte_hbm, wpe_hbm, out_hbm, idx_v, wpe_v,
           b0, b1, b2, b3, gsems, osems):
    bufs = [b0, b1, b2, b3]
    wid = lax.axis_index("s") * NC + lax.axis_index("c")
    col0 = wid * W
    pltpu.sync_copy(x_hbm.at[wid], idx_v)

    def gdesc(b, k):
        return pltpu.make_async_copy(
            wte_hbm.at[idx_v.at[b]], bufs[k], gsems.at[k])

    def odesc(b, k):
        return pltpu.make_async_copy(
            bufs[k], out_hbm.at[pl.ds(b * SEQ + col0, W)], osems.at[k])

    def add_chunk(k):
        @plsc.parallel_loop(0, W, unroll=2)
        def tbody(t):
            for j in range(DSUB):
                sl = pl.ds(j * LANES, LANES)
                plsc.addupdate(bufs[k].at[t, sl], wpe_v[t, sl])

    def step(b, k, first=False, last=False):
        if not last:
            k2 = (k + 2) % NBUF
            if not first:
                odesc(b - 2, k2).wait()
            gdesc(b + 2, k2).start()
        gdesc(b, k).wait()
        add_chunk(k)
        odesc(b, k).start()

    gdesc(0, 0).start()
    gdesc(1, 1).start()
    pltpu.sync_copy(wpe_hbm.at[pl.ds(col0, W)], wpe_v)
    step(0, 0, first=True)
    step(1, 1, first=True)
    step(2, 2)
    step(3, 3)

    def body(i, c):
        b = 4 * i
        for k in range(NBUF):
            step(b + k, k)
        return c

    lax.fori_loop(1, BATCH // NBUF - 1, body, 0)

    for k in range(NBUF):
        b = BATCH - NBUF + k
        step(b, k, last=(b + 2 >= BATCH))
    for k in range(NBUF):
        odesc(BATCH - NBUF + k, k).wait()


def kernel(x, wte, wpe):
    xr = x.astype(jnp.int32).reshape(BATCH, NW, W).transpose(1, 0, 2)
    out = _embed(xr, wte, wpe)
    return out.reshape(BATCH, SEQ, D_MODEL)


# speedup vs baseline: 1.0186x; 1.0186x over previous
import functools

import jax
import jax.numpy as jnp
from jax import lax
from jax.experimental import pallas as pl
from jax.experimental.pallas import tpu as pltpu
from jax.experimental.pallas import tpu_sc as plsc

D_MODEL = 768
MAX_POS = 1024
BATCH = 64
SEQ = 1024

NC = 2
NS = 16
NW = NC * NS

W = SEQ // NW
LANES = 16
DSUB = D_MODEL // LANES
NBUF = 4

_mesh = plsc.VectorSubcoreMesh(core_axis_name="c", subcore_axis_name="s")


@functools.partial(
    pl.kernel,
    mesh=_mesh,
    out_type=jax.ShapeDtypeStruct((BATCH * SEQ, D_MODEL), jnp.float32),
    scratch_types=[
        pltpu.VMEM((BATCH, W), jnp.int32),
        pltpu.VMEM((W, D_MODEL), jnp.float32),
        pltpu.VMEM((W, D_MODEL), jnp.float32),
        pltpu.VMEM((W, D_MODEL), jnp.float32),
        pltpu.VMEM((W, D_MODEL), jnp.float32),
        pltpu.VMEM((W, D_MODEL), jnp.float32),
        pltpu.SemaphoreType.DMA((NBUF,)),
        pltpu.SemaphoreType.DMA((NBUF,)),
    ],
)
def _embed(x_hbm, wte_hbm, wpe_hbm, out_hbm, idx_v, wpe_v,
           b0, b1, b2, b3, gsems, osems):
    bufs = [b0, b1, b2, b3]
    wid = lax.axis_index("s") * NC + lax.axis_index("c")
    col0 = wid * W
    pltpu.sync_copy(x_hbm.at[wid], idx_v)

    def gdesc(b, k):
        return pltpu.make_async_copy(
            wte_hbm.at[idx_v.at[b]], bufs[k], gsems.at[k])

    def odesc(b, k):
        return pltpu.make_async_copy(
            bufs[k], out_hbm.at[pl.ds(b * SEQ + col0, W)], osems.at[k])

    def add_chunk(k):
        def tbody(i, c):
            for u in range(2):
                t = 2 * i + u
                for j in range(DSUB):
                    sl = pl.ds(j * LANES, LANES)
                    plsc.addupdate(bufs[k].at[t, sl], wpe_v[t, sl])
            return c

        lax.fori_loop(0, W // 2, tbody, 0)

    def step(b, k, first=False, last=False):
        if not last:
            k2 = (k + 2) % NBUF
            if not first:
                odesc(b - 2, k2).wait()
            gdesc(b + 2, k2).start()
        gdesc(b, k).wait()
        add_chunk(k)
        odesc(b, k).start()

    gdesc(0, 0).start()
    gdesc(1, 1).start()
    pltpu.sync_copy(wpe_hbm.at[pl.ds(col0, W)], wpe_v)
    step(0, 0, first=True)
    step(1, 1, first=True)
    step(2, 2)
    step(3, 3)

    def body(i, c):
        b = 4 * i
        for k in range(NBUF):
            step(b + k, k)
        return c

    lax.fori_loop(1, BATCH // NBUF - 1, body, 0)

    for k in range(NBUF):
        b = BATCH - NBUF + k
        step(b, k, last=(b + 2 >= BATCH))
    for k in range(NBUF):
        odesc(BATCH - NBUF + k, k).wait()


def kernel(x, wte, wpe):
    xr = x.astype(jnp.int32).reshape(BATCH, NW, W).transpose(1, 0, 2)
    out = _embed(xr, wte, wpe)
    return out.reshape(BATCH, SEQ, D_MODEL)


# speedup vs baseline: 1.0234x; 1.0048x over previous
import functools

import jax
import jax.numpy as jnp
from jax import lax
from jax.experimental import pallas as pl
from jax.experimental.pallas import tpu as pltpu
from jax.experimental.pallas import tpu_sc as plsc

D_MODEL = 768
BATCH = 64
SEQ = 1024

NC = 2
NS = 16
NW = NC * NS

W = SEQ // NW
LANES = 16
DSUB = D_MODEL // LANES
NBUF = 4

_mesh = plsc.VectorSubcoreMesh(core_axis_name="c", subcore_axis_name="s")


@functools.partial(
    pl.kernel,
    mesh=_mesh,
    out_type=jax.ShapeDtypeStruct((BATCH * SEQ, D_MODEL), jnp.float32),
    scratch_types=[
        pltpu.VMEM((BATCH, W), jnp.int32),
        pltpu.VMEM((W, D_MODEL), jnp.float32),
        pltpu.VMEM((W, D_MODEL), jnp.float32),
        pltpu.VMEM((W, D_MODEL), jnp.float32),
        pltpu.VMEM((W, D_MODEL), jnp.float32),
        pltpu.VMEM((W, D_MODEL), jnp.float32),
        pltpu.SemaphoreType.DMA((NBUF,)),
        pltpu.SemaphoreType.DMA((NBUF,)),
    ],
)
def _embed(x_hbm, wte_hbm, wpe_hbm, out_hbm, idx_v, wpe_v,
           b0, b1, b2, b3, gsems, osems):
    bufs = [b0, b1, b2, b3]
    wid = lax.axis_index("s") * NC + lax.axis_index("c")
    col0 = wid * W
    pltpu.sync_copy(x_hbm.at[wid], idx_v)

    def gdesc(b, k):
        return pltpu.make_async_copy(
            wte_hbm.at[idx_v.at[b]], bufs[k], gsems.at[k])

    def odesc(b, k):
        return pltpu.make_async_copy(
            bufs[k], out_hbm.at[pl.ds(b * SEQ + col0, W)], osems.at[k])

    def add_chunk(k):
        def tbody(i, c):
            for u in range(2):
                t = 2 * i + u
                for j in range(DSUB):
                    sl = pl.ds(j * LANES, LANES)
                    plsc.addupdate(bufs[k].at[t, sl], wpe_v[t, sl])
            return c

        lax.fori_loop(0, W // 2, tbody, 0)

    def step(b, k, first=False, last=False):
        if not last:
            k2 = (k + 2) % NBUF
            if not first:
                odesc(b - 2, k2).wait()
            gdesc(b + 2, k2).start()
        gdesc(b, k).wait()
        add_chunk(k)
        odesc(b, k).start()

    gdesc(0, 0).start()
    gdesc(1, 1).start()
    pltpu.sync_copy(wpe_hbm.at[pl.ds(col0, W)], wpe_v)
    step(0, 0, first=True)
    step(1, 1, first=True)
    step(2, 2)
    step(3, 3)

    def body(i, c):
        b = 4 * i
        for k in range(NBUF):
            step(b + k, k)
        return c

    lax.fori_loop(1, BATCH // NBUF - 1, body, 0)

    for k in range(NBUF):
        b = BATCH - NBUF + k
        step(b, k, last=(b + 2 >= BATCH))
    for k in range(NBUF):
        odesc(BATCH - NBUF + k, k).wait()


def kernel(x, wte, wpe):
    xr = x.astype(jnp.int32).reshape(BATCH, NW, W).transpose(1, 0, 2)
    out = _embed(xr, wte, wpe)
    return out.reshape(BATCH, SEQ, D_MODEL)
